# baseline (speedup 1.0000x reference)
import functools

import jax
import jax.numpy as jnp
from jax import lax
from jax.experimental import pallas as pl
from jax.experimental.pallas import tpu as pltpu
from jax.experimental.pallas import tpu_sc as plsc

DIM = 32
NW = 32
CH = 800
NBUF = 4


@functools.partial(jax.jit, static_argnames=("b_total",))
def _sc_gather(ids_flat, table, b_total):
    b_per_w = b_total // NW
    n_chunks = b_per_w // CH
    mesh = plsc.VectorSubcoreMesh(core_axis_name="c", subcore_axis_name="s")

    scratch = (
        [pltpu.VMEM((CH,), jnp.int32) for _ in range(NBUF)]
        + [pltpu.VMEM((CH, DIM), jnp.float32) for _ in range(NBUF)]
        + [pltpu.SemaphoreType.DMA for _ in range(2 * NBUF)]
    )

    @functools.partial(
        pl.kernel,
        mesh=mesh,
        out_type=jax.ShapeDtypeStruct((b_total, DIM), jnp.float32),
        scratch_types=scratch,
        compiler_params=pltpu.CompilerParams(use_tc_tiling_on_sc=False),
    )
    def k(ids_hbm, table_hbm, out_hbm, *bufs):
        idx = bufs[:NBUF]
        rows = bufs[NBUF:2 * NBUF]
        gsem = bufs[2 * NBUF:3 * NBUF]
        ssem = bufs[3 * NBUF:]
        wid = lax.axis_index("s") * 2 + lax.axis_index("c")
        base_w = wid * b_per_w

        g_copy = [None] * NBUF
        s_copy = [None] * NBUF

        for i in range(NBUF):
            pltpu.sync_copy(ids_hbm.at[pl.ds(base_w + i * CH, CH)], idx[i])
            g_copy[i] = pltpu.async_copy(table_hbm.at[idx[i]], rows[i], gsem[i])

        for i in range(n_chunks):
            cur = i % NBUF
            g_copy[cur].wait()
            s_copy[cur] = pltpu.async_copy(
                rows[cur], out_hbm.at[pl.ds(base_w + i * CH, CH)], ssem[cur])
            nxt_chunk = i + NBUF
            if nxt_chunk < n_chunks:
                base_n = base_w + nxt_chunk * CH
                pltpu.sync_copy(ids_hbm.at[pl.ds(base_n, CH)], idx[cur])
                s_copy[cur].wait()
                g_copy[cur] = pltpu.async_copy(
                    table_hbm.at[idx[cur]], rows[cur], gsem[cur])

        for i in range(max(0, n_chunks - NBUF), n_chunks):
            s_copy[i % NBUF].wait()

    return k(ids_flat, table)


def kernel(ids, table):
    b, h = ids.shape
    ids_flat = ids.reshape(-1).astype(jnp.int32)
    out = _sc_gather(ids_flat, table, b * h)
    return out.reshape(b, h, DIM)


# speedup vs baseline: 1.2195x; 1.2195x over previous
import functools

import jax
import jax.numpy as jnp
from jax import lax
from jax.experimental import pallas as pl
from jax.experimental.pallas import tpu as pltpu
from jax.experimental.pallas import tpu_sc as plsc

DIM = 32
NW = 32
LANES = 16
BQ = 16384 // NW
HIST = 50


@jax.jit
def _sc_gather(ids_flat, table):
    mesh = plsc.VectorSubcoreMesh(core_axis_name="c", subcore_axis_name="s")

    @functools.partial(
        pl.kernel,
        mesh=mesh,
        out_type=jax.ShapeDtypeStruct((HIST, DIM, 16384), jnp.float32),
        scratch_types=[
            pltpu.VMEM((BQ,), jnp.int32),
            pltpu.VMEM((BQ, DIM), jnp.float32),
            pltpu.VMEM((DIM, BQ), jnp.float32),
            pltpu.SemaphoreType.DMA,
            pltpu.SemaphoreType.DMA,
        ],
        compiler_params=pltpu.CompilerParams(
            use_tc_tiling_on_sc=False, needs_layout_passes=False),
    )
    def k(ids_hbm, table_hbm, out_hbm, idx_v, rowbuf, colbuf, gsem, wsem):
        wid = lax.axis_index("s") * 2 + lax.axis_index("c")
        b0 = wid * BQ
        iota = lax.iota(jnp.int32, LANES)

        def body(l, carry):
            pltpu.sync_copy(ids_hbm.at[pl.ds(l * 16384 + b0, BQ)], idx_v)
            pltpu.async_copy(table_hbm.at[idx_v], rowbuf, gsem).wait()
            for bb in range(0, BQ, LANES):
                ridx = iota + bb
                for d in range(DIM):
                    cidx = jnp.full((LANES,), d, jnp.int32)
                    v = plsc.load_gather(rowbuf, [ridx, cidx])
                    colbuf[d, pl.ds(bb, LANES)] = v
            cp = pltpu.async_copy(
                colbuf, out_hbm.at[l, :, pl.ds(b0, BQ)], wsem)
            cp.wait()
            return carry

        lax.fori_loop(0, HIST, body, 0)

    return k(ids_flat, table)


def kernel(ids, table):
    ids_flat = jnp.transpose(ids).reshape(-1).astype(jnp.int32)
    out_rm = _sc_gather(ids_flat, table)
    return jnp.transpose(out_rm, (2, 0, 1))


# speedup vs baseline: 1.7592x; 1.4425x over previous
import functools

import jax
import jax.numpy as jnp
from jax import lax
from jax.experimental import pallas as pl
from jax.experimental.pallas import tpu as pltpu
from jax.experimental.pallas import tpu_sc as plsc

DIM = 32
NW = 32
LANES = 16
BQ = 16384 // NW
HIST = 50


@jax.jit
def _sc_gather(ids_flat, table):
    mesh = plsc.VectorSubcoreMesh(core_axis_name="c", subcore_axis_name="s")

    @functools.partial(
        pl.kernel,
        mesh=mesh,
        out_type=jax.ShapeDtypeStruct((HIST, DIM, 16384), jnp.float32),
        scratch_types=[
            pltpu.VMEM((BQ,), jnp.int32),
            pltpu.VMEM((BQ, DIM), jnp.float32),
            pltpu.VMEM((DIM, BQ), jnp.float32),
            pltpu.SemaphoreType.DMA,
            pltpu.SemaphoreType.DMA,
        ],
        compiler_params=pltpu.CompilerParams(
            use_tc_tiling_on_sc=False, needs_layout_passes=False),
    )
    def k(ids_hbm, table_hbm, out_hbm, idx_v, rowbuf, colbuf, gsem, wsem):
        wid = lax.axis_index("s") * 2 + lax.axis_index("c")
        b0 = wid * BQ
        iota = lax.iota(jnp.int32, LANES)
        cols = [
            ((iota + d0) & (LANES - 1)) + LANES * dhi
            for dhi in range(DIM // LANES)
            for d0 in range(LANES)
        ]

        def body(l, carry):
            pltpu.sync_copy(ids_hbm.at[pl.ds(l * 16384 + b0, BQ)], idx_v)
            pltpu.async_copy(table_hbm.at[idx_v], rowbuf, gsem).wait()
            def tbody(bb, tcarry):
                ridx = iota + bb * LANES
                for cidx in cols:
                    v = plsc.load_gather(rowbuf, [ridx, cidx])
                    plsc.store_scatter(colbuf, [cidx, ridx], v)
                return tcarry

            lax.fori_loop(0, BQ // LANES, tbody, 0)
            cp = pltpu.async_copy(
                colbuf, out_hbm.at[l, :, pl.ds(b0, BQ)], wsem)
            cp.wait()
            return carry

        lax.fori_loop(0, HIST, body, 0)

    return k(ids_flat, table)


def kernel(ids, table):
    ids_flat = jnp.transpose(ids).reshape(-1).astype(jnp.int32)
    out_rm = _sc_gather(ids_flat, table)
    return jnp.transpose(out_rm, (2, 0, 1))


# speedup vs baseline: 1.9867x; 1.1293x over previous
import functools

import jax
import jax.numpy as jnp
from jax import lax
from jax.experimental import pallas as pl
from jax.experimental.pallas import tpu as pltpu
from jax.experimental.pallas import tpu_sc as plsc

DIM = 32
NW = 32
LANES = 16
BQ = 16384 // NW
HIST = 50


@jax.jit
def _sc_gather(ids_flat, table):
    mesh = plsc.VectorSubcoreMesh(core_axis_name="c", subcore_axis_name="s")

    @functools.partial(
        pl.kernel,
        mesh=mesh,
        out_type=jax.ShapeDtypeStruct((HIST, DIM, 16384), jnp.float32),
        scratch_types=[
            pltpu.VMEM((BQ,), jnp.int32),
            pltpu.VMEM((BQ,), jnp.int32),
            pltpu.VMEM((BQ, DIM), jnp.float32),
            pltpu.VMEM((BQ, DIM), jnp.float32),
            pltpu.VMEM((DIM, BQ), jnp.float32),
            pltpu.VMEM((DIM, BQ), jnp.float32),
            pltpu.SemaphoreType.DMA,
            pltpu.SemaphoreType.DMA,
            pltpu.SemaphoreType.DMA,
            pltpu.SemaphoreType.DMA,
            pltpu.SemaphoreType.DMA,
            pltpu.SemaphoreType.DMA,
        ],
        compiler_params=pltpu.CompilerParams(
            use_tc_tiling_on_sc=False, needs_layout_passes=False),
    )
    def k(ids_hbm, table_hbm, out_hbm, idx0, idx1, row0, row1, col0, col1,
          i0, i1, g0, g1, w0, w1):
        idxs = (idx0, idx1)
        rows = (row0, row1)
        colb = (col0, col1)
        isem = (i0, i1)
        gsem = (g0, g1)
        wsem = (w0, w1)
        wid = lax.axis_index("s") * 2 + lax.axis_index("c")
        b0 = wid * BQ
        iota = lax.iota(jnp.int32, LANES)
        diag = [
            ((iota + d0) & (LANES - 1)) + LANES * dhi
            for dhi in range(DIM // LANES)
            for d0 in range(LANES)
        ]

        def idx_start(l, p):
            pltpu.async_copy(
                ids_hbm.at[pl.ds(l * 16384 + b0, BQ)], idxs[p], isem[p])

        def idx_wait(p):
            pltpu.make_async_copy(
                ids_hbm.at[pl.ds(b0, BQ)], idxs[p], isem[p]).wait()

        def gather_start(p):
            pltpu.async_copy(table_hbm.at[idxs[p]], rows[p], gsem[p])

        def gather_wait(p):
            pltpu.make_async_copy(
                table_hbm.at[idxs[p]], rows[p], gsem[p]).wait()

        def write_start(l, p):
            pltpu.async_copy(
                colb[p], out_hbm.at[l, :, pl.ds(b0, BQ)], wsem[p])

        def write_wait(p):
            pltpu.make_async_copy(
                colb[p], out_hbm.at[0, :, pl.ds(b0, BQ)], wsem[p]).wait()

        def transpose(p):
            def tbody(bb, tcarry):
                ridx = iota + bb * LANES
                for cidx in diag:
                    v = plsc.load_gather(rows[p], [ridx, cidx])
                    plsc.store_scatter(colb[p], [cidx, ridx], v)
                return tcarry

            lax.fori_loop(0, BQ // LANES, tbody, 0)

        idx_start(0, 0)
        idx_wait(0)
        gather_start(0)
        idx_start(1, 1)
        for l in (0, 1):
            p, q = l % 2, 1 - l % 2
            idx_wait(q)
            gather_start(q)
            gather_wait(p)
            transpose(p)
            write_start(l, p)
            idx_start(l + 2, p)

        def pair_body(pr, carry):
            for sub in range(2):
                l = 2 * pr + sub
                p, q = sub, 1 - sub
                idx_wait(q)
                gather_start(q)
                gather_wait(p)
                write_wait(p)
                transpose(p)
                write_start(l, p)
                idx_start(l + 2, p)
            return carry

        lax.fori_loop(1, HIST // 2 - 1, pair_body, 0)

        idx_wait(1)
        gather_start(1)
        gather_wait(0)
        write_wait(0)
        transpose(0)
        write_start(48, 0)
        gather_wait(1)
        write_wait(1)
        transpose(1)
        write_start(49, 1)
        write_wait(0)
        write_wait(1)

    return k(ids_flat, table)


def kernel(ids, table):
    ids_flat = jnp.transpose(ids).reshape(-1).astype(jnp.int32)
    out_rm = _sc_gather(ids_flat, table)
    return jnp.transpose(out_rm, (2, 0, 1))
